# Initial kernel scaffold; baseline (speedup 1.0000x reference)
import jax
import jax.numpy as jnp
from jax.experimental import pallas as pl


def kernel(node_init, W1, Wself1, b1, relw1, W2, Wself2, b2, relw2, edge_index, edge_type):
    raise NotImplementedError("write your pallas kernel here")


# speedup vs baseline: 4.0491x; 4.0491x over previous
import jax
import jax.numpy as jnp
from jax import lax
from jax.experimental import pallas as pl
from jax.experimental.pallas import tpu as pltpu
from jax.experimental.pallas import tpu_sc as plsc

N = 10000
E = 320000
D = 128
R = 11

NC = 2
NS = 16
NW = NC * NS
CHUNK = 128
EPW = 10112
NCHUNKS = EPW // CHUNK
E_PAD = EPW * NW

SUB_ROWS = 624
REM_ROW0 = NS * SUB_ROWS
ACC_ROWS = N + 16
DEG_W = 128

TBL_BLK = 1000
TBL_NBLK = R * (N // TBL_BLK) + 1
TBL_ROWS = TBL_NBLK * TBL_BLK


def _table_body(x_ref, rw_ref, o_ref):
    o_ref[...] = x_ref[...] * rw_ref[0]


def _build_table(x, relw_pad):
    nb = N // TBL_BLK
    return pl.pallas_call(
        _table_body,
        grid=(TBL_NBLK,),
        in_specs=[
            pl.BlockSpec((TBL_BLK, D), lambda i: (i % nb, 0)),
            pl.BlockSpec((1, 1, D), lambda i: (jnp.minimum(i // nb, R), 0, 0)),
        ],
        out_specs=pl.BlockSpec((TBL_BLK, D), lambda i: (i, 0)),
        out_shape=jax.ShapeDtypeStruct((TBL_ROWS, D), jnp.float32),
    )(x, relw_pad)


def _dense_body(p0, p1, d0, d1, x_ref, w_ref, ws_ref, b_ref, o_ref):
    deg = jnp.maximum(d0[...][:, 0:1] + d1[...][:, 0:1], 1.0)
    agg = (p0[...] + p1[...]) / deg
    acc = (
        jnp.dot(agg, w_ref[...], preferred_element_type=jnp.float32)
        + jnp.dot(x_ref[...], ws_ref[...], preferred_element_type=jnp.float32)
        + b_ref[...]
    )
    o_ref[...] = jnp.maximum(acc, 0.0)


def _dense(p0, p1, d0, d1, x, w, wself, b2d):
    blk = 1000
    nb = N // blk
    return pl.pallas_call(
        _dense_body,
        grid=(nb,),
        in_specs=[
            pl.BlockSpec((blk, D), lambda i: (i, 0)),
            pl.BlockSpec((blk, D), lambda i: (i, 0)),
            pl.BlockSpec((blk, DEG_W), lambda i: (i, 0)),
            pl.BlockSpec((blk, DEG_W), lambda i: (i, 0)),
            pl.BlockSpec((blk, D), lambda i: (i, 0)),
            pl.BlockSpec((D, D), lambda i: (0, 0)),
            pl.BlockSpec((D, D), lambda i: (0, 0)),
            pl.BlockSpec((1, D), lambda i: (0, 0)),
        ],
        out_specs=pl.BlockSpec((blk, D), lambda i: (i, 0)),
        out_shape=jax.ShapeDtypeStruct((N, D), jnp.float32),
    )(p0, p1, d0, d1, x, w, wself, b2d)


_MESH = plsc.VectorSubcoreMesh(
    core_axis_name="c", subcore_axis_name="s", num_cores=NC, num_subcores=NS
)


def _agg_body(tbl, srcp, etp, dstp, agg_out,
              src_b, et_b, dst_b, gidx_b, rows_v, acc_sh, sem):
    c = lax.axis_index("c")
    s = lax.axis_index("s")
    w = c * NS + s
    row0 = s * SUB_ROWS

    @pl.loop(0, CHUNK)
    def _zero_rows(i):
        for g in range(D // 16):
            rows_v[i, pl.ds(g * 16, 16)] = jnp.zeros((16,), jnp.float32)

    @pl.loop(0, 4)
    def _zero_acc(j):
        pltpu.sync_copy(rows_v, acc_sh.at[pl.ds(row0 + j * CHUNK, CHUNK)])

    tail = SUB_ROWS - 4 * CHUNK
    pltpu.sync_copy(rows_v.at[pl.ds(0, tail)],
                    acc_sh.at[pl.ds(row0 + 4 * CHUNK, tail)])

    @pl.when(s == NS - 1)
    def _zero_rem():
        pltpu.sync_copy(rows_v.at[pl.ds(0, ACC_ROWS - REM_ROW0)],
                        acc_sh.at[pl.ds(REM_ROW0, ACC_ROWS - REM_ROW0)])

    plsc.subcore_barrier()

    @pl.loop(0, NCHUNKS)
    def _edges(k):
        base = w * EPW + k * CHUNK
        pltpu.sync_copy(srcp.at[pl.ds(base, CHUNK)], src_b)
        pltpu.sync_copy(etp.at[pl.ds(base, CHUNK)], et_b)
        pltpu.sync_copy(dstp.at[pl.ds(base, CHUNK)], dst_b)
        for g in range(CHUNK // 16):
            sl = pl.ds(g * 16, 16)
            gidx_b[sl] = et_b[sl] * N + src_b[sl]
        pltpu.async_copy(tbl.at[gidx_b], rows_v, sem).wait()
        pltpu.sync_copy(rows_v, acc_sh.at[dst_b], add=True)

    plsc.subcore_barrier()

    pltpu.sync_copy(acc_sh.at[pl.ds(row0, SUB_ROWS)],
                    agg_out.at[c, pl.ds(row0, SUB_ROWS)])

    @pl.when(s == NS - 1)
    def _out_rem():
        pltpu.sync_copy(acc_sh.at[pl.ds(REM_ROW0, N - REM_ROW0)],
                        agg_out.at[c, pl.ds(REM_ROW0, N - REM_ROW0)])


_sc_agg = pl.kernel(
    _agg_body,
    out_type=jax.ShapeDtypeStruct((NC, N, D), jnp.float32),
    mesh=_MESH,
    scratch_types=[
        pltpu.VMEM((CHUNK,), jnp.int32),
        pltpu.VMEM((CHUNK,), jnp.int32),
        pltpu.VMEM((CHUNK,), jnp.int32),
        pltpu.VMEM((CHUNK,), jnp.int32),
        pltpu.VMEM((CHUNK, D), jnp.float32),
        pltpu.VMEM_SHARED((ACC_ROWS, D), jnp.float32),
        pltpu.SemaphoreType.DMA,
    ],
)


def _deg_body(dstp, deg_out, dst_b, ones_v, deg_sh, sem):
    c = lax.axis_index("c")
    s = lax.axis_index("s")
    w = c * NS + s
    row0 = s * SUB_ROWS

    @pl.loop(0, CHUNK)
    def _zero_ones(i):
        for g in range(DEG_W // 16):
            ones_v[i, pl.ds(g * 16, 16)] = jnp.zeros((16,), jnp.float32)

    @pl.loop(0, 4)
    def _zero_deg(j):
        pltpu.sync_copy(ones_v, deg_sh.at[pl.ds(row0 + j * CHUNK, CHUNK)])

    tail = SUB_ROWS - 4 * CHUNK
    pltpu.sync_copy(ones_v.at[pl.ds(0, tail)],
                    deg_sh.at[pl.ds(row0 + 4 * CHUNK, tail)])

    @pl.when(s == NS - 1)
    def _zero_rem():
        pltpu.sync_copy(ones_v.at[pl.ds(0, ACC_ROWS - REM_ROW0)],
                        deg_sh.at[pl.ds(REM_ROW0, ACC_ROWS - REM_ROW0)])

    @pl.loop(0, CHUNK)
    def _set_ones(i):
        for g in range(DEG_W // 16):
            ones_v[i, pl.ds(g * 16, 16)] = jnp.full((16,), 1.0, jnp.float32)

    plsc.subcore_barrier()

    @pl.loop(0, NCHUNKS)
    def _edges(k):
        base = w * EPW + k * CHUNK
        pltpu.sync_copy(dstp.at[pl.ds(base, CHUNK)], dst_b)
        pltpu.sync_copy(ones_v, deg_sh.at[dst_b], add=True)

    plsc.subcore_barrier()

    pltpu.sync_copy(deg_sh.at[pl.ds(row0, SUB_ROWS)],
                    deg_out.at[c, pl.ds(row0, SUB_ROWS)])

    @pl.when(s == NS - 1)
    def _out_rem():
        pltpu.sync_copy(deg_sh.at[pl.ds(REM_ROW0, N - REM_ROW0)],
                        deg_out.at[c, pl.ds(REM_ROW0, N - REM_ROW0)])


_sc_deg = pl.kernel(
    _deg_body,
    out_type=jax.ShapeDtypeStruct((NC, N, DEG_W), jnp.float32),
    mesh=_MESH,
    scratch_types=[
        pltpu.VMEM((CHUNK,), jnp.int32),
        pltpu.VMEM((CHUNK, DEG_W), jnp.float32),
        pltpu.VMEM_SHARED((ACC_ROWS, DEG_W), jnp.float32),
        pltpu.SemaphoreType.DMA,
    ],
)


def kernel(node_init, W1, Wself1, b1, relw1, W2, Wself2, b2, relw2,
           edge_index, edge_type):
    src = edge_index[0]
    dst = edge_index[1]
    pad = E_PAD - E
    srcp = jnp.concatenate([src, jnp.zeros((pad,), jnp.int32)])
    dstp = jnp.concatenate([dst, jnp.full((pad,), N, jnp.int32)])
    etp = jnp.concatenate([edge_type, jnp.full((pad,), R, jnp.int32)])
    zrow = jnp.zeros((1, D), jnp.float32)

    deg = _sc_deg(dstp)

    def layer(x, w, wself, b, relw):
        tbl = _build_table(
            x, jnp.concatenate([relw, zrow], axis=0).reshape(R + 1, 1, D))
        agg = _sc_agg(tbl, srcp, etp, dstp)
        return _dense(agg[0], agg[1], deg[0], deg[1], x, w, wself,
                      b.reshape(1, D))

    h1 = layer(node_init, W1, Wself1, b1, relw1)
    return layer(h1, W2, Wself2, b2, relw2)


# speedup vs baseline: 6.0384x; 1.4913x over previous
import jax
import jax.numpy as jnp
from jax import lax
from jax.experimental import pallas as pl
from jax.experimental.pallas import tpu as pltpu
from jax.experimental.pallas import tpu_sc as plsc

N = 10000
E = 320000
D = 128
R = 11

NC = 2
NS = 16
NW = NC * NS
CHUNK = 128
EPW = 10112
NCHUNKS = EPW // CHUNK
E_PAD = EPW * NW

SUB_ROWS = 624
REM_ROW0 = NS * SUB_ROWS
ACC_ROWS = N + 16
DEG_W = 128

TBL_BLK = 1000
TBL_NBLK = R * (N // TBL_BLK) + 1
TBL_ROWS = TBL_NBLK * TBL_BLK


def _table_body(x_ref, rw_ref, o_ref):
    o_ref[...] = x_ref[...] * rw_ref[0]


def _build_table(x, relw_pad):
    nb = N // TBL_BLK
    return pl.pallas_call(
        _table_body,
        grid=(TBL_NBLK,),
        in_specs=[
            pl.BlockSpec((TBL_BLK, D), lambda i: (i % nb, 0)),
            pl.BlockSpec((1, 1, D), lambda i: (jnp.minimum(i // nb, R), 0, 0)),
        ],
        out_specs=pl.BlockSpec((TBL_BLK, D), lambda i: (i, 0)),
        out_shape=jax.ShapeDtypeStruct((TBL_ROWS, D), jnp.float32),
    )(x, relw_pad)


def _dense_body(p0, p1, d0, d1, x_ref, w_ref, ws_ref, b_ref, o_ref):
    deg = jnp.maximum(d0[...][:, 0:1] + d1[...][:, 0:1], 1.0)
    agg = (p0[...] + p1[...]) / deg
    acc = (
        jnp.dot(agg, w_ref[...], preferred_element_type=jnp.float32)
        + jnp.dot(x_ref[...], ws_ref[...], preferred_element_type=jnp.float32)
        + b_ref[...]
    )
    o_ref[...] = jnp.maximum(acc, 0.0)


def _dense(p0, p1, d0, d1, x, w, wself, b2d):
    blk = 1000
    nb = N // blk
    return pl.pallas_call(
        _dense_body,
        grid=(nb,),
        in_specs=[
            pl.BlockSpec((blk, D), lambda i: (i, 0)),
            pl.BlockSpec((blk, D), lambda i: (i, 0)),
            pl.BlockSpec((blk, DEG_W), lambda i: (i, 0)),
            pl.BlockSpec((blk, DEG_W), lambda i: (i, 0)),
            pl.BlockSpec((blk, D), lambda i: (i, 0)),
            pl.BlockSpec((D, D), lambda i: (0, 0)),
            pl.BlockSpec((D, D), lambda i: (0, 0)),
            pl.BlockSpec((1, D), lambda i: (0, 0)),
        ],
        out_specs=pl.BlockSpec((blk, D), lambda i: (i, 0)),
        out_shape=jax.ShapeDtypeStruct((N, D), jnp.float32),
    )(p0, p1, d0, d1, x, w, wself, b2d)


_MESH = plsc.VectorSubcoreMesh(
    core_axis_name="c", subcore_axis_name="s", num_cores=NC, num_subcores=NS
)


PASS0 = (NCHUNKS + 1) // 2
PASS_SIZES = (PASS0, NCHUNKS - PASS0)


def _agg_body(tbl, srcp, etp, dstp, agg_out,
              src_a, et_a, dst_a, rows0, rows1, acc_sh, sem_i, sem0, sem1):
    c = lax.axis_index("c")
    s = lax.axis_index("s")
    w = c * NS + s
    row0 = s * SUB_ROWS

    @pl.loop(0, CHUNK)
    def _zero_rows(i):
        for g in range(D // 16):
            rows0[i, pl.ds(g * 16, 16)] = jnp.zeros((16,), jnp.float32)

    @pl.loop(0, 4)
    def _zero_acc(j):
        pltpu.sync_copy(rows0, acc_sh.at[pl.ds(row0 + j * CHUNK, CHUNK)])

    tail = SUB_ROWS - 4 * CHUNK
    pltpu.sync_copy(rows0.at[pl.ds(0, tail)],
                    acc_sh.at[pl.ds(row0 + 4 * CHUNK, tail)])

    @pl.when(s == NS - 1)
    def _zero_rem():
        pltpu.sync_copy(rows0.at[pl.ds(0, N - REM_ROW0)],
                        acc_sh.at[pl.ds(REM_ROW0, N - REM_ROW0)])

    plsc.subcore_barrier()

    for p, n_k in enumerate(PASS_SIZES):
        k0 = p * PASS0
        pltpu.async_copy(srcp.at[w, pl.ds(k0, n_k)], src_a.at[pl.ds(0, n_k)],
                         sem_i)
        pltpu.async_copy(etp.at[w, pl.ds(k0, n_k)], et_a.at[pl.ds(0, n_k)],
                         sem_i)
        pltpu.async_copy(dstp.at[w, pl.ds(k0, n_k)], dst_a.at[pl.ds(0, n_k)],
                         sem_i)
        pltpu.make_async_copy(srcp.at[w, pl.ds(k0, n_k)],
                              src_a.at[pl.ds(0, n_k)], sem_i).wait()
        pltpu.make_async_copy(etp.at[w, pl.ds(k0, n_k)],
                              et_a.at[pl.ds(0, n_k)], sem_i).wait()
        pltpu.make_async_copy(dstp.at[w, pl.ds(k0, n_k)],
                              dst_a.at[pl.ds(0, n_k)], sem_i).wait()

        @pl.loop(0, n_k)
        def _gidx(k):
            for g in range(CHUNK // 16):
                sl = pl.ds(g * 16, 16)
                src_a[k, sl] = et_a[k, sl] * N + src_a[k, sl]

        pltpu.async_copy(tbl.at[src_a.at[0]], rows0, sem0)
        pltpu.async_copy(tbl.at[src_a.at[1]], rows1, sem1)

        @pl.loop(0, n_k // 2)
        def _edges(k2):
            k = k2 * 2
            pltpu.make_async_copy(tbl.at[src_a.at[k]], rows0, sem0).wait()
            pltpu.sync_copy(rows0, acc_sh.at[dst_a.at[k]], add=True)

            @pl.when(k + 2 < n_k)
            def _next_even():
                pltpu.async_copy(tbl.at[src_a.at[k + 2]], rows0, sem0)

            pltpu.make_async_copy(tbl.at[src_a.at[k + 1]], rows1, sem1).wait()
            pltpu.sync_copy(rows1, acc_sh.at[dst_a.at[k + 1]], add=True)

            @pl.when(k + 3 < n_k)
            def _next_odd():
                pltpu.async_copy(tbl.at[src_a.at[k + 3]], rows1, sem1)

        if n_k % 2:
            k_last = n_k - 1
            pltpu.make_async_copy(tbl.at[src_a.at[k_last]], rows0,
                                  sem0).wait()
            pltpu.sync_copy(rows0, acc_sh.at[dst_a.at[k_last]], add=True)

    plsc.subcore_barrier()

    pltpu.sync_copy(acc_sh.at[pl.ds(row0, SUB_ROWS)],
                    agg_out.at[c, pl.ds(row0, SUB_ROWS)])

    @pl.when(s == NS - 1)
    def _out_rem():
        pltpu.sync_copy(acc_sh.at[pl.ds(REM_ROW0, N - REM_ROW0)],
                        agg_out.at[c, pl.ds(REM_ROW0, N - REM_ROW0)])


_sc_agg = pl.kernel(
    _agg_body,
    out_type=jax.ShapeDtypeStruct((NC, N, D), jnp.float32),
    mesh=_MESH,
    scratch_types=[
        pltpu.VMEM((PASS0, CHUNK), jnp.int32),
        pltpu.VMEM((PASS0, CHUNK), jnp.int32),
        pltpu.VMEM((PASS0, CHUNK), jnp.int32),
        pltpu.VMEM((CHUNK, D), jnp.float32),
        pltpu.VMEM((CHUNK, D), jnp.float32),
        pltpu.VMEM_SHARED((N, D), jnp.float32),
        pltpu.SemaphoreType.DMA,
        pltpu.SemaphoreType.DMA,
        pltpu.SemaphoreType.DMA,
    ],
)


def _deg_body(dstp, deg_out, dst_a, ones_v, deg_sh, sem):
    c = lax.axis_index("c")
    s = lax.axis_index("s")
    w = c * NS + s
    row0 = s * SUB_ROWS

    pltpu.async_copy(dstp.at[w], dst_a, sem)

    @pl.loop(0, CHUNK)
    def _zero_ones(i):
        for g in range(DEG_W // 16):
            ones_v[i, pl.ds(g * 16, 16)] = jnp.zeros((16,), jnp.float32)

    @pl.loop(0, 4)
    def _zero_deg(j):
        pltpu.sync_copy(ones_v, deg_sh.at[pl.ds(row0 + j * CHUNK, CHUNK)])

    tail = SUB_ROWS - 4 * CHUNK
    pltpu.sync_copy(ones_v.at[pl.ds(0, tail)],
                    deg_sh.at[pl.ds(row0 + 4 * CHUNK, tail)])

    @pl.when(s == NS - 1)
    def _zero_rem():
        pltpu.sync_copy(ones_v.at[pl.ds(0, ACC_ROWS - REM_ROW0)],
                        deg_sh.at[pl.ds(REM_ROW0, ACC_ROWS - REM_ROW0)])

    @pl.loop(0, CHUNK)
    def _set_ones(i):
        for g in range(DEG_W // 16):
            ones_v[i, pl.ds(g * 16, 16)] = jnp.full((16,), 1.0, jnp.float32)

    pltpu.make_async_copy(dstp.at[w], dst_a, sem).wait()
    plsc.subcore_barrier()

    @pl.loop(0, NCHUNKS)
    def _edges(k):
        pltpu.sync_copy(ones_v, deg_sh.at[dst_a.at[k]], add=True)

    plsc.subcore_barrier()

    pltpu.sync_copy(deg_sh.at[pl.ds(row0, SUB_ROWS)],
                    deg_out.at[c, pl.ds(row0, SUB_ROWS)])

    @pl.when(s == NS - 1)
    def _out_rem():
        pltpu.sync_copy(deg_sh.at[pl.ds(REM_ROW0, N - REM_ROW0)],
                        deg_out.at[c, pl.ds(REM_ROW0, N - REM_ROW0)])


_sc_deg = pl.kernel(
    _deg_body,
    out_type=jax.ShapeDtypeStruct((NC, N, DEG_W), jnp.float32),
    mesh=_MESH,
    scratch_types=[
        pltpu.VMEM((NCHUNKS, CHUNK), jnp.int32),
        pltpu.VMEM((CHUNK, DEG_W), jnp.float32),
        pltpu.VMEM_SHARED((ACC_ROWS, DEG_W), jnp.float32),
        pltpu.SemaphoreType.DMA,
    ],
)


def kernel(node_init, W1, Wself1, b1, relw1, W2, Wself2, b2, relw2,
           edge_index, edge_type):
    src = edge_index[0]
    dst = edge_index[1]
    pad = E_PAD - E
    shp = (NW, NCHUNKS, CHUNK)
    srcp = jnp.concatenate([src, jnp.zeros((pad,), jnp.int32)]).reshape(shp)
    dstp = jnp.concatenate([dst, jnp.zeros((pad,), jnp.int32)]).reshape(shp)
    dstp_deg = jnp.concatenate(
        [dst, jnp.full((pad,), N, jnp.int32)]).reshape(shp)
    etp = jnp.concatenate(
        [edge_type, jnp.full((pad,), R, jnp.int32)]).reshape(shp)
    zrow = jnp.zeros((1, D), jnp.float32)

    deg = _sc_deg(dstp_deg)

    def layer(x, w, wself, b, relw):
        tbl = _build_table(
            x, jnp.concatenate([relw, zrow], axis=0).reshape(R + 1, 1, D))
        agg = _sc_agg(tbl, srcp, etp, dstp)
        return _dense(agg[0], agg[1], deg[0], deg[1], x, w, wself,
                      b.reshape(1, D))

    h1 = layer(node_init, W1, Wself1, b1, relw1)
    return layer(h1, W2, Wself2, b2, relw2)
